# Initial kernel scaffold; baseline (speedup 1.0000x reference)
import jax
import jax.numpy as jnp
from jax.experimental import pallas as pl


def kernel(x, edge_index, edge_attr, batch, W_in, b_in, We, be, W1, b1, g1, bt1, W2, b2, g2, bt2, Wc1, bc1, gc, btc, Wc2, bc2):
    raise NotImplementedError("write your pallas kernel here")


# speedup vs baseline: 2.1884x; 2.1884x over previous
import functools

import jax
import jax.numpy as jnp
from jax import lax
from jax.experimental import pallas as pl
from jax.experimental.pallas import tpu as pltpu
from jax.experimental.pallas import tpu_sc as plsc

N = 10000
E = 320000
D_IN = 128
H = 256
HH = 128
ED = 16
NLAYERS = 3
G = 64
C = 10
JK = H * NLAYERS

NS = 16
K = 128
CHUNKS = E // K
ZROWS = 624
ZTAIL = N - NS * ZROWS

_SC_MESH = plsc.VectorSubcoreMesh(core_axis_name="c", subcore_axis_name="s")
_NEG_INF = float("-inf")


def _proj_body(x_ref, w_ref, b_ref, lo_ref, hi_ref):
    acc = jnp.dot(x_ref[...], w_ref[...], preferred_element_type=jnp.float32)
    acc = acc + b_ref[...]
    lo_ref[...] = acc[:, :HH]
    hi_ref[...] = acc[:, HH:]


def _input_proj(x, w, b):
    R = 400
    return pl.pallas_call(
        _proj_body,
        grid=(N // R,),
        in_specs=[
            pl.BlockSpec((R, D_IN), lambda i: (i, 0)),
            pl.BlockSpec((D_IN, H), lambda i: (0, 0)),
            pl.BlockSpec((1, H), lambda i: (0, 0)),
        ],
        out_specs=[
            pl.BlockSpec((R, HH), lambda i: (i, 0)),
            pl.BlockSpec((R, HH), lambda i: (i, 0)),
        ],
        out_shape=[jax.ShapeDtypeStruct((N, HH), jnp.float32)] * 2,
    )(x, w, b.reshape(1, H))


def _ea_proj(edge_attr, we, be):
    R = 2000
    return pl.pallas_call(
        _proj_body,
        grid=(E // R,),
        in_specs=[
            pl.BlockSpec((R, ED), lambda i: (i, 0)),
            pl.BlockSpec((ED, H), lambda i: (0, 0)),
            pl.BlockSpec((1, H), lambda i: (0, 0)),
        ],
        out_specs=[
            pl.BlockSpec((R, HH), lambda i: (i, 0)),
            pl.BlockSpec((R, HH), lambda i: (i, 0)),
        ],
        out_shape=[jax.ShapeDtypeStruct((E, HH), jnp.float32)] * 2,
    )(edge_attr, we, be.reshape(1, H))


def _mlp_body(hlo_ref, hhi_ref, alo_ref, ahi_ref, w1_ref, b1_ref, w2_ref,
              b2_ref, olo_ref, ohi_ref):
    z = jnp.concatenate(
        [hlo_ref[...] + alo_ref[...], hhi_ref[...] + ahi_ref[...]], axis=1)
    y = jnp.dot(z, w1_ref[...], preferred_element_type=jnp.float32)
    y = jnp.maximum(y + b1_ref[...], 0.0)
    o = jnp.dot(y, w2_ref[...], preferred_element_type=jnp.float32)
    o = jnp.maximum(o + b2_ref[...], 0.0)
    olo_ref[...] = o[:, :HH]
    ohi_ref[...] = o[:, HH:]


def _mlp(h_lo, h_hi, a_lo, a_hi, w1, b1, w2, b2):
    R = 400
    blk = pl.BlockSpec((R, HH), lambda i: (i, 0))
    wblk = pl.BlockSpec((H, H), lambda i: (0, 0))
    bblk = pl.BlockSpec((1, H), lambda i: (0, 0))
    return pl.pallas_call(
        _mlp_body,
        grid=(N // R,),
        in_specs=[blk, blk, blk, blk, wblk, bblk, wblk, bblk],
        out_specs=[blk, blk],
        out_shape=[jax.ShapeDtypeStruct((N, HH), jnp.float32)] * 2,
    )(h_lo, h_hi, a_lo, a_hi, w1, b1.reshape(1, H), w2, b2.reshape(1, H))


def _maxpool_body(b_ref, h1l, h1h, h2l, h2h, h3l, h3h, out_ref, cnt_ref):
    i = pl.program_id(0)

    @pl.when(i == 0)
    def _():
        out_ref[...] = jnp.full((G, JK), _NEG_INF, jnp.float32)
        cnt_ref[...] = jnp.zeros((G, 128), jnp.float32)

    bvec = b_ref[...]
    jkb = jnp.concatenate(
        [h1l[...], h1h[...], h2l[...], h2h[...], h3l[...], h3h[...]], axis=1)
    for g in range(G):
        mask = bvec == g

        @pl.when(jnp.any(mask))
        def _(g=g):
            m = jnp.max(jnp.where(mask, jkb, _NEG_INF), axis=0)
            out_ref[g, :] = jnp.maximum(out_ref[g, :], m)
            cnt_ref[g, :] = cnt_ref[g, :] + jnp.sum(mask.astype(jnp.float32))


def _maxpool(batch2d, hs):
    R = 400
    blk = pl.BlockSpec((R, HH), lambda i: (i, 0))
    return pl.pallas_call(
        _maxpool_body,
        grid=(N // R,),
        in_specs=[pl.BlockSpec((R, 1), lambda i: (i, 0))] + [blk] * 6,
        out_specs=[pl.BlockSpec((G, JK), lambda i: (0, 0)),
                   pl.BlockSpec((G, 128), lambda i: (0, 0))],
        out_shape=[jax.ShapeDtypeStruct((G, JK), jnp.float32),
                   jax.ShapeDtypeStruct((G, 128), jnp.float32)],
    )(batch2d, *hs)


def _head_body(sum_ref, mx_ref, cnt_ref, w1_ref, b1_ref, w2_ref, b2_ref,
               out_ref):
    s = sum_ref[0] + sum_ref[1]
    mx = mx_ref[...]
    mx = jnp.where(mx == _NEG_INF, 0.0, mx)
    cnt = cnt_ref[:, :1]
    mean = s / jnp.maximum(cnt, 1.0)
    z = jnp.concatenate([mean, mx], axis=1)
    y = jnp.dot(z, w1_ref[...], preferred_element_type=jnp.float32)
    y = jnp.maximum(y + b1_ref[...], 0.0)
    o = jnp.dot(y, w2_ref[...], preferred_element_type=jnp.float32)
    out_ref[...] = o + b2_ref[...]


def _head(sum2, mx, cnt2, wc1, bc1, wc2, bc2):
    return pl.pallas_call(
        _head_body,
        out_shape=jax.ShapeDtypeStruct((G, C), jnp.float32),
    )(sum2, mx, cnt2, wc1, bc1.reshape(1, H), wc2, bc2.reshape(1, C))


@functools.partial(
    pl.kernel,
    out_type=[jax.ShapeDtypeStruct((N, HH), jnp.float32),
              jax.ShapeDtypeStruct((N, HH), jnp.float32)],
    mesh=_SC_MESH,
    scratch_types=[
        pltpu.VMEM((K,), jnp.int32),
        pltpu.VMEM((K,), jnp.int32),
        pltpu.VMEM((K, HH), jnp.float32),
        pltpu.VMEM((K, HH), jnp.float32),
        pltpu.VMEM_SHARED((N, HH), jnp.float32),
        pltpu.SemaphoreType.DMA,
    ],
)
def _edge_aggr(src_hbm, dst_hbm, h_lo, h_hi, ea_lo, ea_hi, zeros_hbm,
               out_lo, out_hi, src_v, dst_v, h_buf, ea_buf, aggr_sh, sem):
    c = lax.axis_index("c")
    s = lax.axis_index("s")
    row0 = s * ZROWS

    pltpu.sync_copy(zeros_hbm, aggr_sh.at[pl.ds(row0, ZROWS)])

    @pl.when(s == NS - 1)
    def _():
        pltpu.sync_copy(zeros_hbm.at[pl.ds(0, ZTAIL)],
                        aggr_sh.at[pl.ds(NS * ZROWS, ZTAIL)])

    plsc.subcore_barrier()

    def run(h_ref, ea_ref):
        nchunks = (CHUNKS + NS - 1 - s) // NS

        def chunk_body(i, carry):
            base = (s + i * NS) * K
            pltpu.sync_copy(src_hbm.at[pl.ds(base, K)], src_v)
            pltpu.sync_copy(dst_hbm.at[pl.ds(base, K)], dst_v)
            pltpu.async_copy(h_ref.at[src_v], h_buf, sem).wait()
            pltpu.sync_copy(ea_ref.at[pl.ds(base, K)], ea_buf)

            def row_body(r, rc):
                for j in range(HH // 16):
                    sl = pl.ds(j * 16, 16)
                    h_buf[r, sl] = jnp.maximum(h_buf[r, sl] + ea_buf[r, sl],
                                               0.0)
                return rc

            lax.fori_loop(0, K, row_body, 0)
            pltpu.sync_copy(h_buf, aggr_sh.at[dst_v], add=True)
            return carry

        lax.fori_loop(0, nchunks, chunk_body, 0)

    @pl.when(c == 0)
    def _():
        run(h_lo, ea_lo)

    @pl.when(c == 1)
    def _():
        run(h_hi, ea_hi)

    plsc.subcore_barrier()

    def copy_out(out_ref):
        pltpu.sync_copy(aggr_sh.at[pl.ds(row0, ZROWS)],
                        out_ref.at[pl.ds(row0, ZROWS)])

        @pl.when(s == NS - 1)
        def _():
            pltpu.sync_copy(aggr_sh.at[pl.ds(NS * ZROWS, ZTAIL)],
                            out_ref.at[pl.ds(NS * ZROWS, ZTAIL)])

    @pl.when(c == 0)
    def _():
        copy_out(out_lo)

    @pl.when(c == 1)
    def _():
        copy_out(out_hi)


_POOL_CH = 16
_POOL_NCHUNK = N // _POOL_CH


@functools.partial(
    pl.kernel,
    out_type=jax.ShapeDtypeStruct((2, G, JK), jnp.float32),
    mesh=_SC_MESH,
    scratch_types=[
        [pltpu.VMEM((_POOL_CH, HH), jnp.float32) for _ in range(6)],
        pltpu.VMEM((_POOL_CH,), jnp.int32),
        [pltpu.VMEM_SHARED((G, HH), jnp.float32) for _ in range(6)],
    ],
)
def _pool(batch_hbm, h1_lo, h1_hi, h2_lo, h2_hi, h3_lo, h3_hi, z128_hbm,
          out_sum, rbufs, bbuf, ssum):
    c = lax.axis_index("c")
    s = lax.axis_index("s")
    w = c * NS + s
    h_refs = (h1_lo, h1_hi, h2_lo, h2_hi, h3_lo, h3_hi)

    @pl.when(s == 0)
    def _():
        for a in range(6):
            pltpu.sync_copy(z128_hbm, ssum[a])

    plsc.subcore_barrier()

    nchunks = (_POOL_NCHUNK + 2 * NS - 1 - w) // (2 * NS)

    def chunk_body(i, carry):
        base = (w + i * 2 * NS) * _POOL_CH
        pltpu.sync_copy(batch_hbm.at[pl.ds(base, _POOL_CH)], bbuf)
        for a in range(6):
            pltpu.sync_copy(h_refs[a].at[pl.ds(base, _POOL_CH)], rbufs[a])
            pltpu.sync_copy(rbufs[a], ssum[a].at[bbuf], add=True)
        return carry

    lax.fori_loop(0, nchunks, chunk_body, 0)
    plsc.subcore_barrier()

    for a in range(6):

        @pl.when(s == a)
        def _(a=a):
            pltpu.sync_copy(ssum[a], out_sum.at[c, :, pl.ds(a * HH, HH)])


def kernel(x, edge_index, edge_attr, batch, W_in, b_in, We, be, W1, b1, g1,
           bt1, W2, b2, g2, bt2, Wc1, bc1, gc, btc, Wc2, bc2):
    isr = 1.0 / jnp.sqrt(jnp.float32(1.0 + 1e-5))
    s1 = g1 * isr
    w1f = W1 * s1[:, None, :]
    b1f = b1 * s1 + bt1
    s2 = g2 * isr
    w2f = W2 * s2[:, None, :]
    b2f = b2 * s2 + bt2
    sc = gc * isr
    wc1f = Wc1 * sc[None, :]
    bc1f = bc1 * sc + btc

    src = edge_index[0]
    dst = edge_index[1]
    zeros = jnp.zeros((ZROWS, HH), jnp.float32)
    z128 = jnp.zeros((G, HH), jnp.float32)

    h_lo, h_hi = _input_proj(x, W_in, b_in)
    hs = []
    for l in range(NLAYERS):
        ea_lo, ea_hi = _ea_proj(edge_attr, We[l], be[l])
        a_lo, a_hi = _edge_aggr(src, dst, h_lo, h_hi, ea_lo, ea_hi, zeros)
        h_lo, h_hi = _mlp(h_lo, h_hi, a_lo, a_hi, w1f[l], b1f[l], w2f[l],
                          b2f[l])
        hs += [h_lo, h_hi]

    sum2 = _pool(batch, *hs, z128)
    mx, cnt = _maxpool(batch.reshape(N, 1), hs)
    return _head(sum2, mx, cnt, wc1f, bc1f, Wc2, bc2)


# speedup vs baseline: 3.0259x; 1.3827x over previous
import functools

import jax
import jax.numpy as jnp
from jax import lax
from jax.experimental import pallas as pl
from jax.experimental.pallas import tpu as pltpu
from jax.experimental.pallas import tpu_sc as plsc

N = 10000
E = 320000
D_IN = 128
H = 256
HH = 128
ED = 16
NLAYERS = 3
G = 64
C = 10
JK = H * NLAYERS

NS = 16
K = 80
CHUNKS = E // K
CPT = CHUNKS // NS
ZROWS = 624
ZTAIL = N - NS * ZROWS

_SC_MESH = plsc.VectorSubcoreMesh(core_axis_name="c", subcore_axis_name="s")
_NEG_INF = float("-inf")


def _proj_body(x_ref, w_ref, b_ref, lo_ref, hi_ref):
    acc = jnp.dot(x_ref[...], w_ref[...], preferred_element_type=jnp.float32)
    acc = acc + b_ref[...]
    lo_ref[...] = acc[:, :HH]
    hi_ref[...] = acc[:, HH:]


def _input_proj(x, w, b):
    R = 400
    return pl.pallas_call(
        _proj_body,
        grid=(N // R,),
        in_specs=[
            pl.BlockSpec((R, D_IN), lambda i: (i, 0)),
            pl.BlockSpec((D_IN, H), lambda i: (0, 0)),
            pl.BlockSpec((1, H), lambda i: (0, 0)),
        ],
        out_specs=[
            pl.BlockSpec((R, HH), lambda i: (i, 0)),
            pl.BlockSpec((R, HH), lambda i: (i, 0)),
        ],
        out_shape=[jax.ShapeDtypeStruct((N, HH), jnp.float32)] * 2,
    )(x, w, b.reshape(1, H))


def _ea_proj(edge_attr, we, be):
    R = 2000
    return pl.pallas_call(
        _proj_body,
        grid=(E // R,),
        in_specs=[
            pl.BlockSpec((R, ED), lambda i: (i, 0)),
            pl.BlockSpec((ED, H), lambda i: (0, 0)),
            pl.BlockSpec((1, H), lambda i: (0, 0)),
        ],
        out_specs=[
            pl.BlockSpec((R, HH), lambda i: (i, 0)),
            pl.BlockSpec((R, HH), lambda i: (i, 0)),
        ],
        out_shape=[jax.ShapeDtypeStruct((E, HH), jnp.float32)] * 2,
    )(edge_attr, we, be.reshape(1, H))


def _mlp_body(hlo_ref, hhi_ref, alo_ref, ahi_ref, w1_ref, b1_ref, w2_ref,
              b2_ref, olo_ref, ohi_ref):
    z = jnp.concatenate(
        [hlo_ref[...] + alo_ref[...], hhi_ref[...] + ahi_ref[...]], axis=1)
    y = jnp.dot(z, w1_ref[...], preferred_element_type=jnp.float32)
    y = jnp.maximum(y + b1_ref[...], 0.0)
    o = jnp.dot(y, w2_ref[...], preferred_element_type=jnp.float32)
    o = jnp.maximum(o + b2_ref[...], 0.0)
    olo_ref[...] = o[:, :HH]
    ohi_ref[...] = o[:, HH:]


def _mlp(h_lo, h_hi, a_lo, a_hi, w1, b1, w2, b2):
    R = 400
    blk = pl.BlockSpec((R, HH), lambda i: (i, 0))
    wblk = pl.BlockSpec((H, H), lambda i: (0, 0))
    bblk = pl.BlockSpec((1, H), lambda i: (0, 0))
    return pl.pallas_call(
        _mlp_body,
        grid=(N // R,),
        in_specs=[blk, blk, blk, blk, wblk, bblk, wblk, bblk],
        out_specs=[blk, blk],
        out_shape=[jax.ShapeDtypeStruct((N, HH), jnp.float32)] * 2,
    )(h_lo, h_hi, a_lo, a_hi, w1, b1.reshape(1, H), w2, b2.reshape(1, H))


def _maxpool_body(b_ref, h1l, h1h, h2l, h2h, h3l, h3h, out_ref, cnt_ref):
    i = pl.program_id(0)

    @pl.when(i == 0)
    def _():
        out_ref[...] = jnp.full((G, JK), _NEG_INF, jnp.float32)
        cnt_ref[...] = jnp.zeros((G, 128), jnp.float32)

    bvec = b_ref[...]
    jkb = jnp.concatenate(
        [h1l[...], h1h[...], h2l[...], h2h[...], h3l[...], h3h[...]], axis=1)
    for g in range(G):
        mask = bvec == g

        @pl.when(jnp.any(mask))
        def _(g=g):
            m = jnp.max(jnp.where(mask, jkb, _NEG_INF), axis=0)
            out_ref[g, :] = jnp.maximum(out_ref[g, :], m)
            cnt_ref[g, :] = cnt_ref[g, :] + jnp.sum(mask.astype(jnp.float32))


def _maxpool(batch2d, hs):
    R = 400
    blk = pl.BlockSpec((R, HH), lambda i: (i, 0))
    return pl.pallas_call(
        _maxpool_body,
        grid=(N // R,),
        in_specs=[pl.BlockSpec((R, 1), lambda i: (i, 0))] + [blk] * 6,
        out_specs=[pl.BlockSpec((G, JK), lambda i: (0, 0)),
                   pl.BlockSpec((G, 128), lambda i: (0, 0))],
        out_shape=[jax.ShapeDtypeStruct((G, JK), jnp.float32),
                   jax.ShapeDtypeStruct((G, 128), jnp.float32)],
    )(batch2d, *hs)


def _head_body(sum_ref, mx_ref, cnt_ref, w1_ref, b1_ref, w2_ref, b2_ref,
               out_ref):
    s = sum_ref[0] + sum_ref[1]
    mx = mx_ref[...]
    mx = jnp.where(mx == _NEG_INF, 0.0, mx)
    cnt = cnt_ref[:, :1]
    mean = s / jnp.maximum(cnt, 1.0)
    z = jnp.concatenate([mean, mx], axis=1)
    y = jnp.dot(z, w1_ref[...], preferred_element_type=jnp.float32)
    y = jnp.maximum(y + b1_ref[...], 0.0)
    o = jnp.dot(y, w2_ref[...], preferred_element_type=jnp.float32)
    out_ref[...] = o + b2_ref[...]


def _head(sum2, mx, cnt2, wc1, bc1, wc2, bc2):
    return pl.pallas_call(
        _head_body,
        out_shape=jax.ShapeDtypeStruct((G, C), jnp.float32),
    )(sum2, mx, cnt2, wc1, bc1.reshape(1, H), wc2, bc2.reshape(1, C))


@functools.partial(
    pl.kernel,
    out_type=[jax.ShapeDtypeStruct((N, HH), jnp.float32),
              jax.ShapeDtypeStruct((N, HH), jnp.float32)],
    mesh=_SC_MESH,
    scratch_types=[
        [pltpu.VMEM((K,), jnp.int32) for _ in range(2)],
        [pltpu.VMEM((K,), jnp.int32) for _ in range(2)],
        [pltpu.VMEM((K, HH), jnp.float32) for _ in range(2)],
        [pltpu.VMEM((K, HH), jnp.float32) for _ in range(2)],
        pltpu.VMEM_SHARED((N, HH), jnp.float32),
        [pltpu.SemaphoreType.DMA for _ in range(2)],
        [pltpu.SemaphoreType.DMA for _ in range(2)],
    ],
)
def _edge_aggr(src_hbm, dst_hbm, h_lo, h_hi, ea_lo, ea_hi, zeros_hbm,
               out_lo, out_hi, src_vs, dst_vs, h_bufs, ea_bufs, aggr_sh,
               gsems, esems):
    c = lax.axis_index("c")
    s = lax.axis_index("s")
    row0 = s * ZROWS

    pltpu.sync_copy(zeros_hbm, aggr_sh.at[pl.ds(row0, ZROWS)])

    @pl.when(s == NS - 1)
    def _():
        pltpu.sync_copy(zeros_hbm.at[pl.ds(0, ZTAIL)],
                        aggr_sh.at[pl.ds(NS * ZROWS, ZTAIL)])

    plsc.subcore_barrier()

    def run(h_ref, ea_ref):

        def idx_and_fire(j, b):
            base = (s + j * NS) * K
            pltpu.sync_copy(src_hbm.at[pl.ds(base, K)], src_vs[b])
            pltpu.sync_copy(dst_hbm.at[pl.ds(base, K)], dst_vs[b])
            pltpu.async_copy(h_ref.at[src_vs[b]], h_bufs[b], gsems[b])
            pltpu.async_copy(ea_ref.at[pl.ds(base, K)], ea_bufs[b],
                             esems[b])

        idx_and_fire(0, 0)

        def pair_body(p, carry):
            for b in range(2):
                j = 2 * p + b

                @pl.when(j + 1 < CPT)
                def _(b=b, j=j):
                    idx_and_fire(j + 1, 1 - b)

                base = (s + j * NS) * K
                pltpu.make_async_copy(h_ref.at[src_vs[b]], h_bufs[b],
                                      gsems[b]).wait()
                pltpu.make_async_copy(ea_ref.at[pl.ds(base, K)],
                                      ea_bufs[b], esems[b]).wait()

                def row_body(r, rc):
                    for jj in range(HH // 16):
                        sl = pl.ds(jj * 16, 16)
                        h_bufs[b][r, sl] = jnp.maximum(
                            h_bufs[b][r, sl] + ea_bufs[b][r, sl], 0.0)
                    return rc

                lax.fori_loop(0, K, row_body, 0)
                pltpu.sync_copy(h_bufs[b], aggr_sh.at[dst_vs[b]], add=True)
            return carry

        lax.fori_loop(0, CPT // 2, pair_body, 0)

    @pl.when(c == 0)
    def _():
        run(h_lo, ea_lo)

    @pl.when(c == 1)
    def _():
        run(h_hi, ea_hi)

    plsc.subcore_barrier()

    def copy_out(out_ref):
        pltpu.sync_copy(aggr_sh.at[pl.ds(row0, ZROWS)],
                        out_ref.at[pl.ds(row0, ZROWS)])

        @pl.when(s == NS - 1)
        def _():
            pltpu.sync_copy(aggr_sh.at[pl.ds(NS * ZROWS, ZTAIL)],
                            out_ref.at[pl.ds(NS * ZROWS, ZTAIL)])

    @pl.when(c == 0)
    def _():
        copy_out(out_lo)

    @pl.when(c == 1)
    def _():
        copy_out(out_hi)


_POOL_CH = 16
_POOL_NCHUNK = N // _POOL_CH


@functools.partial(
    pl.kernel,
    out_type=jax.ShapeDtypeStruct((2, G, JK), jnp.float32),
    mesh=_SC_MESH,
    scratch_types=[
        [pltpu.VMEM((_POOL_CH, HH), jnp.float32) for _ in range(6)],
        pltpu.VMEM((_POOL_CH,), jnp.int32),
        [pltpu.VMEM_SHARED((G, HH), jnp.float32) for _ in range(6)],
    ],
)
def _pool(batch_hbm, h1_lo, h1_hi, h2_lo, h2_hi, h3_lo, h3_hi, z128_hbm,
          out_sum, rbufs, bbuf, ssum):
    c = lax.axis_index("c")
    s = lax.axis_index("s")
    w = c * NS + s
    h_refs = (h1_lo, h1_hi, h2_lo, h2_hi, h3_lo, h3_hi)

    @pl.when(s == 0)
    def _():
        for a in range(6):
            pltpu.sync_copy(z128_hbm, ssum[a])

    plsc.subcore_barrier()

    nchunks = (_POOL_NCHUNK + 2 * NS - 1 - w) // (2 * NS)

    def chunk_body(i, carry):
        base = (w + i * 2 * NS) * _POOL_CH
        pltpu.sync_copy(batch_hbm.at[pl.ds(base, _POOL_CH)], bbuf)
        for a in range(6):
            pltpu.sync_copy(h_refs[a].at[pl.ds(base, _POOL_CH)], rbufs[a])
            pltpu.sync_copy(rbufs[a], ssum[a].at[bbuf], add=True)
        return carry

    lax.fori_loop(0, nchunks, chunk_body, 0)
    plsc.subcore_barrier()

    for a in range(6):

        @pl.when(s == a)
        def _(a=a):
            pltpu.sync_copy(ssum[a], out_sum.at[c, :, pl.ds(a * HH, HH)])


def kernel(x, edge_index, edge_attr, batch, W_in, b_in, We, be, W1, b1, g1,
           bt1, W2, b2, g2, bt2, Wc1, bc1, gc, btc, Wc2, bc2):
    isr = 1.0 / jnp.sqrt(jnp.float32(1.0 + 1e-5))
    s1 = g1 * isr
    w1f = W1 * s1[:, None, :]
    b1f = b1 * s1 + bt1
    s2 = g2 * isr
    w2f = W2 * s2[:, None, :]
    b2f = b2 * s2 + bt2
    sc = gc * isr
    wc1f = Wc1 * sc[None, :]
    bc1f = bc1 * sc + btc

    src = edge_index[0]
    dst = edge_index[1]
    zeros = jnp.zeros((ZROWS, HH), jnp.float32)
    z128 = jnp.zeros((G, HH), jnp.float32)

    h_lo, h_hi = _input_proj(x, W_in, b_in)
    hs = []
    for l in range(NLAYERS):
        ea_lo, ea_hi = _ea_proj(edge_attr, We[l], be[l])
        a_lo, a_hi = _edge_aggr(src, dst, h_lo, h_hi, ea_lo, ea_hi, zeros)
        h_lo, h_hi = _mlp(h_lo, h_hi, a_lo, a_hi, w1f[l], b1f[l], w2f[l],
                          b2f[l])
        hs += [h_lo, h_hi]

    sum2 = _pool(batch, *hs, z128)
    mx, cnt = _maxpool(batch.reshape(N, 1), hs)
    return _head(sum2, mx, cnt, wc1f, bc1f, Wc2, bc2)


# speedup vs baseline: 3.3828x; 1.1179x over previous
import functools

import jax
import jax.numpy as jnp
from jax import lax
from jax.experimental import pallas as pl
from jax.experimental.pallas import tpu as pltpu
from jax.experimental.pallas import tpu_sc as plsc

N = 10000
E = 320000
D_IN = 128
H = 256
HH = 128
ED = 16
NLAYERS = 3
G = 64
C = 10
JK = H * NLAYERS

NS = 16
K = 80
CHUNKS = E // K
CPT = CHUNKS // NS
ZROWS = 624
ZTAIL = N - NS * ZROWS

_SC_MESH = plsc.VectorSubcoreMesh(core_axis_name="c", subcore_axis_name="s")
_NEG_INF = float("-inf")


def _proj_body(x_ref, w_ref, b_ref, lo_ref, hi_ref):
    acc = jnp.dot(x_ref[...], w_ref[...], preferred_element_type=jnp.float32)
    acc = acc + b_ref[...]
    lo_ref[...] = acc[:, :HH]
    hi_ref[...] = acc[:, HH:]


def _input_proj(x, w, b):
    R = 400
    return pl.pallas_call(
        _proj_body,
        grid=(N // R,),
        in_specs=[
            pl.BlockSpec((R, D_IN), lambda i: (i, 0)),
            pl.BlockSpec((D_IN, H), lambda i: (0, 0)),
            pl.BlockSpec((1, H), lambda i: (0, 0)),
        ],
        out_specs=[
            pl.BlockSpec((R, HH), lambda i: (i, 0)),
            pl.BlockSpec((R, HH), lambda i: (i, 0)),
        ],
        out_shape=[jax.ShapeDtypeStruct((N, HH), jnp.float32)] * 2,
    )(x, w, b.reshape(1, H))


def _ea_proj(edge_attr, we, be):
    R = 2000
    return pl.pallas_call(
        _proj_body,
        grid=(E // R,),
        in_specs=[
            pl.BlockSpec((R, ED), lambda i: (i, 0)),
            pl.BlockSpec((ED, H), lambda i: (0, 0)),
            pl.BlockSpec((1, H), lambda i: (0, 0)),
        ],
        out_specs=[
            pl.BlockSpec((R, HH), lambda i: (i, 0)),
            pl.BlockSpec((R, HH), lambda i: (i, 0)),
        ],
        out_shape=[jax.ShapeDtypeStruct((E, HH), jnp.float32)] * 2,
    )(edge_attr, we, be.reshape(1, H))


def _mlp_body(hlo_ref, hhi_ref, alo_ref, ahi_ref, w1_ref, b1_ref, w2_ref,
              b2_ref, olo_ref, ohi_ref):
    z = jnp.concatenate(
        [hlo_ref[...] + alo_ref[...], hhi_ref[...] + ahi_ref[...]], axis=1)
    y = jnp.dot(z, w1_ref[...], preferred_element_type=jnp.float32)
    y = jnp.maximum(y + b1_ref[...], 0.0)
    o = jnp.dot(y, w2_ref[...], preferred_element_type=jnp.float32)
    o = jnp.maximum(o + b2_ref[...], 0.0)
    olo_ref[...] = o[:, :HH]
    ohi_ref[...] = o[:, HH:]


def _mlp(h_lo, h_hi, a_lo, a_hi, w1, b1, w2, b2):
    R = 400
    blk = pl.BlockSpec((R, HH), lambda i: (i, 0))
    wblk = pl.BlockSpec((H, H), lambda i: (0, 0))
    bblk = pl.BlockSpec((1, H), lambda i: (0, 0))
    return pl.pallas_call(
        _mlp_body,
        grid=(N // R,),
        in_specs=[blk, blk, blk, blk, wblk, bblk, wblk, bblk],
        out_specs=[blk, blk],
        out_shape=[jax.ShapeDtypeStruct((N, HH), jnp.float32)] * 2,
    )(h_lo, h_hi, a_lo, a_hi, w1, b1.reshape(1, H), w2, b2.reshape(1, H))


def _maxpool_body(b_ref, h1l, h1h, h2l, h2h, h3l, h3h, out_ref, cnt_ref):
    i = pl.program_id(0)

    @pl.when(i == 0)
    def _():
        out_ref[...] = jnp.full((G, JK), _NEG_INF, jnp.float32)
        cnt_ref[...] = jnp.zeros((G, 128), jnp.float32)

    bvec = b_ref[...]
    jkb = jnp.concatenate(
        [h1l[...], h1h[...], h2l[...], h2h[...], h3l[...], h3h[...]], axis=1)
    for g in range(G):
        mask = bvec == g

        @pl.when(jnp.any(mask))
        def _(g=g):
            m = jnp.max(jnp.where(mask, jkb, _NEG_INF), axis=0)
            out_ref[g, :] = jnp.maximum(out_ref[g, :], m)
            cnt_ref[g, :] = cnt_ref[g, :] + jnp.sum(mask.astype(jnp.float32))


def _maxpool(batch2d, hs):
    R = 400
    blk = pl.BlockSpec((R, HH), lambda i: (i, 0))
    return pl.pallas_call(
        _maxpool_body,
        grid=(N // R,),
        in_specs=[pl.BlockSpec((R, 1), lambda i: (i, 0))] + [blk] * 6,
        out_specs=[pl.BlockSpec((G, JK), lambda i: (0, 0)),
                   pl.BlockSpec((G, 128), lambda i: (0, 0))],
        out_shape=[jax.ShapeDtypeStruct((G, JK), jnp.float32),
                   jax.ShapeDtypeStruct((G, 128), jnp.float32)],
    )(batch2d, *hs)


def _head_body(sum_ref, mx_ref, cnt_ref, w1_ref, b1_ref, w2_ref, b2_ref,
               out_ref):
    s = sum_ref[0] + sum_ref[1]
    mx = mx_ref[...]
    mx = jnp.where(mx == _NEG_INF, 0.0, mx)
    cnt = cnt_ref[:, :1]
    mean = s / jnp.maximum(cnt, 1.0)
    z = jnp.concatenate([mean, mx], axis=1)
    y = jnp.dot(z, w1_ref[...], preferred_element_type=jnp.float32)
    y = jnp.maximum(y + b1_ref[...], 0.0)
    o = jnp.dot(y, w2_ref[...], preferred_element_type=jnp.float32)
    out_ref[...] = o + b2_ref[...]


def _head(sum2, mx, cnt2, wc1, bc1, wc2, bc2):
    return pl.pallas_call(
        _head_body,
        out_shape=jax.ShapeDtypeStruct((G, C), jnp.float32),
    )(sum2, mx, cnt2, wc1, bc1.reshape(1, H), wc2, bc2.reshape(1, C))


@functools.partial(
    pl.kernel,
    out_type=[jax.ShapeDtypeStruct((N, HH), jnp.float32),
              jax.ShapeDtypeStruct((N, HH), jnp.float32)],
    mesh=_SC_MESH,
    scratch_types=[
        [pltpu.VMEM((2, K), jnp.int32) for _ in range(2)],
        [pltpu.VMEM((K, HH), jnp.float32) for _ in range(2)],
        [pltpu.VMEM((K, HH), jnp.float32) for _ in range(2)],
        pltpu.VMEM_SHARED((N, HH), jnp.float32),
        [pltpu.SemaphoreType.DMA for _ in range(2)],
        [pltpu.SemaphoreType.DMA for _ in range(2)],
    ],
)
def _edge_aggr(ei_hbm, h_lo, h_hi, ea_lo, ea_hi, zeros_hbm,
               out_lo, out_hi, idx_bufs, h_bufs, ea_bufs, aggr_sh,
               gsems, esems):
    c = lax.axis_index("c")
    s = lax.axis_index("s")
    row0 = s * ZROWS

    pltpu.sync_copy(zeros_hbm, aggr_sh.at[pl.ds(row0, ZROWS)])

    @pl.when(s == NS - 1)
    def _():
        pltpu.sync_copy(zeros_hbm.at[pl.ds(0, ZTAIL)],
                        aggr_sh.at[pl.ds(NS * ZROWS, ZTAIL)])

    plsc.subcore_barrier()

    def run(h_ref, ea_ref):

        def idx_and_fire(j, b):
            base = (s + j * NS) * K
            pltpu.sync_copy(ei_hbm.at[s + j * NS], idx_bufs[b])
            pltpu.async_copy(h_ref.at[idx_bufs[b].at[0]], h_bufs[b],
                             gsems[b])
            pltpu.async_copy(ea_ref.at[pl.ds(base, K)], ea_bufs[b],
                             esems[b])

        idx_and_fire(0, 0)

        def pair_body(p, carry):
            for b in range(2):
                j = 2 * p + b

                @pl.when(j + 1 < CPT)
                def _(b=b, j=j):
                    idx_and_fire(j + 1, 1 - b)

                base = (s + j * NS) * K
                pltpu.make_async_copy(h_ref.at[idx_bufs[b].at[0]],
                                      h_bufs[b], gsems[b]).wait()
                pltpu.make_async_copy(ea_ref.at[pl.ds(base, K)],
                                      ea_bufs[b], esems[b]).wait()

                def row_body(r2, rc):
                    for u in range(2):
                        r = r2 * 2 + u
                        for jj in range(HH // 16):
                            sl = pl.ds(jj * 16, 16)
                            h_bufs[b][r, sl] = jnp.maximum(
                                h_bufs[b][r, sl] + ea_bufs[b][r, sl], 0.0)
                    return rc

                lax.fori_loop(0, K // 2, row_body, 0)
                pltpu.sync_copy(h_bufs[b], aggr_sh.at[idx_bufs[b].at[1]],
                                add=True)
            return carry

        lax.fori_loop(0, CPT // 2, pair_body, 0)

    @pl.when(c == 0)
    def _():
        run(h_lo, ea_lo)

    @pl.when(c == 1)
    def _():
        run(h_hi, ea_hi)

    plsc.subcore_barrier()

    def copy_out(out_ref):
        pltpu.sync_copy(aggr_sh.at[pl.ds(row0, ZROWS)],
                        out_ref.at[pl.ds(row0, ZROWS)])

        @pl.when(s == NS - 1)
        def _():
            pltpu.sync_copy(aggr_sh.at[pl.ds(NS * ZROWS, ZTAIL)],
                            out_ref.at[pl.ds(NS * ZROWS, ZTAIL)])

    @pl.when(c == 0)
    def _():
        copy_out(out_lo)

    @pl.when(c == 1)
    def _():
        copy_out(out_hi)


_POOL_CH = 16
_POOL_NCHUNK = N // _POOL_CH


@functools.partial(
    pl.kernel,
    out_type=jax.ShapeDtypeStruct((2, G, JK), jnp.float32),
    mesh=_SC_MESH,
    scratch_types=[
        [pltpu.VMEM((_POOL_CH, HH), jnp.float32) for _ in range(6)],
        pltpu.VMEM((_POOL_CH,), jnp.int32),
        [pltpu.VMEM_SHARED((G, HH), jnp.float32) for _ in range(6)],
    ],
)
def _pool(batch_hbm, h1_lo, h1_hi, h2_lo, h2_hi, h3_lo, h3_hi, z128_hbm,
          out_sum, rbufs, bbuf, ssum):
    c = lax.axis_index("c")
    s = lax.axis_index("s")
    w = c * NS + s
    h_refs = (h1_lo, h1_hi, h2_lo, h2_hi, h3_lo, h3_hi)

    @pl.when(s == 0)
    def _():
        for a in range(6):
            pltpu.sync_copy(z128_hbm, ssum[a])

    plsc.subcore_barrier()

    nchunks = (_POOL_NCHUNK + 2 * NS - 1 - w) // (2 * NS)

    def chunk_body(i, carry):
        base = (w + i * 2 * NS) * _POOL_CH
        pltpu.sync_copy(batch_hbm.at[pl.ds(base, _POOL_CH)], bbuf)
        for a in range(6):
            pltpu.sync_copy(h_refs[a].at[pl.ds(base, _POOL_CH)], rbufs[a])
            pltpu.sync_copy(rbufs[a], ssum[a].at[bbuf], add=True)
        return carry

    lax.fori_loop(0, nchunks, chunk_body, 0)
    plsc.subcore_barrier()

    for a in range(6):

        @pl.when(s == a)
        def _(a=a):
            pltpu.sync_copy(ssum[a], out_sum.at[c, :, pl.ds(a * HH, HH)])


def kernel(x, edge_index, edge_attr, batch, W_in, b_in, We, be, W1, b1, g1,
           bt1, W2, b2, g2, bt2, Wc1, bc1, gc, btc, Wc2, bc2):
    isr = 1.0 / jnp.sqrt(jnp.float32(1.0 + 1e-5))
    s1 = g1 * isr
    w1f = W1 * s1[:, None, :]
    b1f = b1 * s1 + bt1
    s2 = g2 * isr
    w2f = W2 * s2[:, None, :]
    b2f = b2 * s2 + bt2
    sc = gc * isr
    wc1f = Wc1 * sc[None, :]
    bc1f = bc1 * sc + btc

    ei_chunks = jnp.transpose(edge_index.reshape(2, CHUNKS, K), (1, 0, 2))
    zeros = jnp.zeros((ZROWS, HH), jnp.float32)
    z128 = jnp.zeros((G, HH), jnp.float32)

    h_lo, h_hi = _input_proj(x, W_in, b_in)
    hs = []
    for l in range(NLAYERS):
        ea_lo, ea_hi = _ea_proj(edge_attr, We[l], be[l])
        a_lo, a_hi = _edge_aggr(ei_chunks, h_lo, h_hi, ea_lo, ea_hi, zeros)
        h_lo, h_hi = _mlp(h_lo, h_hi, a_lo, a_hi, w1f[l], b1f[l], w2f[l],
                          b2f[l])
        hs += [h_lo, h_hi]

    sum2 = _pool(batch, *hs, z128)
    mx, cnt = _maxpool(batch.reshape(N, 1), hs)
    return _head(sum2, mx, cnt, wc1f, bc1f, Wc2, bc2)


# speedup vs baseline: 3.6254x; 1.0717x over previous
import functools

import jax
import jax.numpy as jnp
from jax import lax
from jax.experimental import pallas as pl
from jax.experimental.pallas import tpu as pltpu
from jax.experimental.pallas import tpu_sc as plsc

N = 10000
E = 320000
D_IN = 128
H = 256
HH = 128
ED = 16
NLAYERS = 3
G = 64
C = 10
JK = H * NLAYERS

NS = 16
K = 80
CHUNKS = E // K
CPT = CHUNKS // NS
ZROWS = 624
ZTAIL = N - NS * ZROWS

_SC_MESH = plsc.VectorSubcoreMesh(core_axis_name="c", subcore_axis_name="s")
_NEG_INF = float("-inf")


def _proj_body(x_ref, w_ref, b_ref, lo_ref, hi_ref):
    acc = jnp.dot(x_ref[...], w_ref[...], preferred_element_type=jnp.float32)
    acc = acc + b_ref[...]
    lo_ref[...] = acc[:, :HH]
    hi_ref[...] = acc[:, HH:]


def _input_proj(x, w, b):
    R = 400
    return pl.pallas_call(
        _proj_body,
        grid=(N // R,),
        in_specs=[
            pl.BlockSpec((R, D_IN), lambda i: (i, 0)),
            pl.BlockSpec((D_IN, H), lambda i: (0, 0)),
            pl.BlockSpec((1, H), lambda i: (0, 0)),
        ],
        out_specs=[
            pl.BlockSpec((R, HH), lambda i: (i, 0)),
            pl.BlockSpec((R, HH), lambda i: (i, 0)),
        ],
        out_shape=[jax.ShapeDtypeStruct((N, HH), jnp.float32)] * 2,
    )(x, w, b.reshape(1, H))


def _ea_proj(edge_attr, we, be):
    R = 2000
    return pl.pallas_call(
        _proj_body,
        grid=(E // R,),
        in_specs=[
            pl.BlockSpec((R, ED), lambda i: (i, 0)),
            pl.BlockSpec((ED, H), lambda i: (0, 0)),
            pl.BlockSpec((1, H), lambda i: (0, 0)),
        ],
        out_specs=[
            pl.BlockSpec((R, HH), lambda i: (i, 0)),
            pl.BlockSpec((R, HH), lambda i: (i, 0)),
        ],
        out_shape=[jax.ShapeDtypeStruct((E, HH), jnp.float32)] * 2,
    )(edge_attr, we, be.reshape(1, H))


def _mlp_body(hlo_ref, hhi_ref, alo_ref, ahi_ref, w1_ref, b1_ref, w2_ref,
              b2_ref, olo_ref, ohi_ref):
    z = jnp.concatenate(
        [hlo_ref[...] + alo_ref[...], hhi_ref[...] + ahi_ref[...]], axis=1)
    y = jnp.dot(z, w1_ref[...], preferred_element_type=jnp.float32)
    y = jnp.maximum(y + b1_ref[...], 0.0)
    o = jnp.dot(y, w2_ref[...], preferred_element_type=jnp.float32)
    o = jnp.maximum(o + b2_ref[...], 0.0)
    olo_ref[...] = o[:, :HH]
    ohi_ref[...] = o[:, HH:]


def _mlp(h_lo, h_hi, a_lo, a_hi, w1, b1, w2, b2):
    R = 400
    blk = pl.BlockSpec((R, HH), lambda i: (i, 0))
    wblk = pl.BlockSpec((H, H), lambda i: (0, 0))
    bblk = pl.BlockSpec((1, H), lambda i: (0, 0))
    return pl.pallas_call(
        _mlp_body,
        grid=(N // R,),
        in_specs=[blk, blk, blk, blk, wblk, bblk, wblk, bblk],
        out_specs=[blk, blk],
        out_shape=[jax.ShapeDtypeStruct((N, HH), jnp.float32)] * 2,
    )(h_lo, h_hi, a_lo, a_hi, w1, b1.reshape(1, H), w2, b2.reshape(1, H))


def _maxpool_body(b_ref, h1l, h1h, h2l, h2h, h3l, h3h, out_ref, cnt_ref):
    i = pl.program_id(0)

    @pl.when(i == 0)
    def _():
        out_ref[...] = jnp.full((G, JK), _NEG_INF, jnp.float32)
        cnt_ref[...] = jnp.zeros((G, 128), jnp.float32)

    bvec = b_ref[...]
    jkb = jnp.concatenate(
        [h1l[...], h1h[...], h2l[...], h2h[...], h3l[...], h3h[...]], axis=1)
    for g in range(G):
        mask = bvec == g

        @pl.when(jnp.any(mask))
        def _(g=g):
            m = jnp.max(jnp.where(mask, jkb, _NEG_INF), axis=0)
            out_ref[g, :] = jnp.maximum(out_ref[g, :], m)
            cnt_ref[g, :] = cnt_ref[g, :] + jnp.sum(mask.astype(jnp.float32))


def _maxpool(batch2d, hs):
    R = 400
    blk = pl.BlockSpec((R, HH), lambda i: (i, 0))
    return pl.pallas_call(
        _maxpool_body,
        grid=(N // R,),
        in_specs=[pl.BlockSpec((R, 1), lambda i: (i, 0))] + [blk] * 6,
        out_specs=[pl.BlockSpec((G, JK), lambda i: (0, 0)),
                   pl.BlockSpec((G, 128), lambda i: (0, 0))],
        out_shape=[jax.ShapeDtypeStruct((G, JK), jnp.float32),
                   jax.ShapeDtypeStruct((G, 128), jnp.float32)],
    )(batch2d, *hs)


def _head_body(sum_ref, mx_ref, cnt_ref, w1_ref, b1_ref, w2_ref, b2_ref,
               out_ref):
    s = sum_ref[0] + sum_ref[1]
    mx = mx_ref[...]
    mx = jnp.where(mx == _NEG_INF, 0.0, mx)
    cnt = cnt_ref[:, :1]
    mean = s / jnp.maximum(cnt, 1.0)
    z = jnp.concatenate([mean, mx], axis=1)
    y = jnp.dot(z, w1_ref[...], preferred_element_type=jnp.float32)
    y = jnp.maximum(y + b1_ref[...], 0.0)
    o = jnp.dot(y, w2_ref[...], preferred_element_type=jnp.float32)
    out_ref[...] = o + b2_ref[...]


def _head(sum2, mx, cnt2, wc1, bc1, wc2, bc2):
    return pl.pallas_call(
        _head_body,
        out_shape=jax.ShapeDtypeStruct((G, C), jnp.float32),
    )(sum2, mx, cnt2, wc1, bc1.reshape(1, H), wc2, bc2.reshape(1, C))


@functools.partial(
    pl.kernel,
    out_type=[jax.ShapeDtypeStruct((N, HH), jnp.float32),
              jax.ShapeDtypeStruct((N, HH), jnp.float32)],
    mesh=_SC_MESH,
    scratch_types=[
        [pltpu.VMEM((2, K), jnp.int32) for _ in range(2)],
        [pltpu.VMEM((K,), jnp.int32) for _ in range(2)],
        [pltpu.VMEM((K, HH), jnp.float32) for _ in range(2)],
        [pltpu.VMEM((K, HH), jnp.float32) for _ in range(2)],
        pltpu.VMEM_SHARED((N, HH), jnp.float32),
        [pltpu.SemaphoreType.DMA for _ in range(2)],
        [pltpu.SemaphoreType.DMA for _ in range(2)],
        [pltpu.SemaphoreType.DMA for _ in range(2)],
    ],
)
def _edge_aggr(ei_hbm, h_lo, h_hi, ea_lo, ea_hi, zeros_hbm,
               out_lo, out_hi, idx_bufs, dst_bufs, h_bufs, ea_bufs, aggr_sh,
               gsems, esems, ssems):
    c = lax.axis_index("c")
    s = lax.axis_index("s")
    row0 = s * ZROWS

    pltpu.sync_copy(zeros_hbm, aggr_sh.at[pl.ds(row0, ZROWS)])

    @pl.when(s == NS - 1)
    def _():
        pltpu.sync_copy(zeros_hbm.at[pl.ds(0, ZTAIL)],
                        aggr_sh.at[pl.ds(NS * ZROWS, ZTAIL)])

    plsc.subcore_barrier()

    def run(h_ref, ea_ref):

        def scatter_wait(b):
            pltpu.make_async_copy(h_bufs[b], aggr_sh.at[dst_bufs[b]],
                                  ssems[b]).wait()

        def idx_and_fire(j, b):
            base = (s + j * NS) * K
            pltpu.sync_copy(ei_hbm.at[s + j * NS], idx_bufs[b])

            @pl.when(j >= 2)
            def _():
                scatter_wait(b)

            pltpu.async_copy(h_ref.at[idx_bufs[b].at[0]], h_bufs[b],
                             gsems[b])
            pltpu.async_copy(ea_ref.at[pl.ds(base, K)], ea_bufs[b],
                             esems[b])

        idx_and_fire(0, 0)

        def pair_body(p, carry):
            for b in range(2):
                j = 2 * p + b

                @pl.when(j + 1 < CPT)
                def _(b=b, j=j):
                    idx_and_fire(j + 1, 1 - b)

                base = (s + j * NS) * K
                pltpu.make_async_copy(h_ref.at[idx_bufs[b].at[0]],
                                      h_bufs[b], gsems[b]).wait()
                pltpu.make_async_copy(ea_ref.at[pl.ds(base, K)],
                                      ea_bufs[b], esems[b]).wait()

                def row_body(r2, rc):
                    for u in range(2):
                        r = r2 * 2 + u
                        for jj in range(HH // 16):
                            sl = pl.ds(jj * 16, 16)
                            h_bufs[b][r, sl] = jnp.maximum(
                                h_bufs[b][r, sl] + ea_bufs[b][r, sl], 0.0)
                    return rc

                lax.fori_loop(0, K // 2, row_body, 0)
                for q in range(K // 16):
                    qsl = pl.ds(q * 16, 16)
                    dst_bufs[b][qsl] = idx_bufs[b][1, qsl]
                pltpu.async_copy(h_bufs[b], aggr_sh.at[dst_bufs[b]],
                                 ssems[b], add=True)
            return carry

        lax.fori_loop(0, CPT // 2, pair_body, 0)
        scatter_wait(0)
        scatter_wait(1)

    @pl.when(c == 0)
    def _():
        run(h_lo, ea_lo)

    @pl.when(c == 1)
    def _():
        run(h_hi, ea_hi)

    plsc.subcore_barrier()

    def copy_out(out_ref):
        pltpu.sync_copy(aggr_sh.at[pl.ds(row0, ZROWS)],
                        out_ref.at[pl.ds(row0, ZROWS)])

        @pl.when(s == NS - 1)
        def _():
            pltpu.sync_copy(aggr_sh.at[pl.ds(NS * ZROWS, ZTAIL)],
                            out_ref.at[pl.ds(NS * ZROWS, ZTAIL)])

    @pl.when(c == 0)
    def _():
        copy_out(out_lo)

    @pl.when(c == 1)
    def _():
        copy_out(out_hi)


_POOL_CH = 16
_POOL_NCHUNK = N // _POOL_CH


@functools.partial(
    pl.kernel,
    out_type=jax.ShapeDtypeStruct((2, G, JK), jnp.float32),
    mesh=_SC_MESH,
    scratch_types=[
        [pltpu.VMEM((_POOL_CH, HH), jnp.float32) for _ in range(6)],
        pltpu.VMEM((_POOL_CH,), jnp.int32),
        [pltpu.VMEM_SHARED((G, HH), jnp.float32) for _ in range(6)],
    ],
)
def _pool(batch_hbm, h1_lo, h1_hi, h2_lo, h2_hi, h3_lo, h3_hi, z128_hbm,
          out_sum, rbufs, bbuf, ssum):
    c = lax.axis_index("c")
    s = lax.axis_index("s")
    w = c * NS + s
    h_refs = (h1_lo, h1_hi, h2_lo, h2_hi, h3_lo, h3_hi)

    @pl.when(s == 0)
    def _():
        for a in range(6):
            pltpu.sync_copy(z128_hbm, ssum[a])

    plsc.subcore_barrier()

    nchunks = (_POOL_NCHUNK + 2 * NS - 1 - w) // (2 * NS)

    def chunk_body(i, carry):
        base = (w + i * 2 * NS) * _POOL_CH
        pltpu.sync_copy(batch_hbm.at[pl.ds(base, _POOL_CH)], bbuf)
        for a in range(6):
            pltpu.sync_copy(h_refs[a].at[pl.ds(base, _POOL_CH)], rbufs[a])
            pltpu.sync_copy(rbufs[a], ssum[a].at[bbuf], add=True)
        return carry

    lax.fori_loop(0, nchunks, chunk_body, 0)
    plsc.subcore_barrier()

    for a in range(6):

        @pl.when(s == a)
        def _(a=a):
            pltpu.sync_copy(ssum[a], out_sum.at[c, :, pl.ds(a * HH, HH)])


def kernel(x, edge_index, edge_attr, batch, W_in, b_in, We, be, W1, b1, g1,
           bt1, W2, b2, g2, bt2, Wc1, bc1, gc, btc, Wc2, bc2):
    isr = 1.0 / jnp.sqrt(jnp.float32(1.0 + 1e-5))
    s1 = g1 * isr
    w1f = W1 * s1[:, None, :]
    b1f = b1 * s1 + bt1
    s2 = g2 * isr
    w2f = W2 * s2[:, None, :]
    b2f = b2 * s2 + bt2
    sc = gc * isr
    wc1f = Wc1 * sc[None, :]
    bc1f = bc1 * sc + btc

    ei_chunks = jnp.transpose(edge_index.reshape(2, CHUNKS, K), (1, 0, 2))
    zeros = jnp.zeros((ZROWS, HH), jnp.float32)
    z128 = jnp.zeros((G, HH), jnp.float32)

    h_lo, h_hi = _input_proj(x, W_in, b_in)
    hs = []
    for l in range(NLAYERS):
        ea_lo, ea_hi = _ea_proj(edge_attr, We[l], be[l])
        a_lo, a_hi = _edge_aggr(ei_chunks, h_lo, h_hi, ea_lo, ea_hi, zeros)
        h_lo, h_hi = _mlp(h_lo, h_hi, a_lo, a_hi, w1f[l], b1f[l], w2f[l],
                          b2f[l])
        hs += [h_lo, h_hi]

    sum2 = _pool(batch, *hs, z128)
    mx, cnt = _maxpool(batch.reshape(N, 1), hs)
    return _head(sum2, mx, cnt, wc1f, bc1f, Wc2, bc2)


# speedup vs baseline: 3.6263x; 1.0003x over previous
import functools

import jax
import jax.numpy as jnp
from jax import lax
from jax.experimental import pallas as pl
from jax.experimental.pallas import tpu as pltpu
from jax.experimental.pallas import tpu_sc as plsc

N = 10000
E = 320000
D_IN = 128
H = 256
HH = 128
ED = 16
NLAYERS = 3
G = 64
C = 10
JK = H * NLAYERS

NS = 16
K = 80
CHUNKS = E // K
CPT = CHUNKS // NS
ZROWS = 624
ZTAIL = N - NS * ZROWS

_SC_MESH = plsc.VectorSubcoreMesh(core_axis_name="c", subcore_axis_name="s")
_NEG_INF = float("-inf")


def _proj_body(x_ref, w_ref, b_ref, lo_ref, hi_ref):
    acc = jnp.dot(x_ref[...], w_ref[...], preferred_element_type=jnp.float32)
    acc = acc + b_ref[...]
    lo_ref[...] = acc[:, :HH]
    hi_ref[...] = acc[:, HH:]


def _input_proj(x, w, b):
    R = 400
    return pl.pallas_call(
        _proj_body,
        grid=(N // R,),
        in_specs=[
            pl.BlockSpec((R, D_IN), lambda i: (i, 0)),
            pl.BlockSpec((D_IN, H), lambda i: (0, 0)),
            pl.BlockSpec((1, H), lambda i: (0, 0)),
        ],
        out_specs=[
            pl.BlockSpec((R, HH), lambda i: (i, 0)),
            pl.BlockSpec((R, HH), lambda i: (i, 0)),
        ],
        out_shape=[jax.ShapeDtypeStruct((N, HH), jnp.float32)] * 2,
    )(x, w, b.reshape(1, H))


def _ea_proj(edge_attr, we, be):
    R = 2000
    return pl.pallas_call(
        _proj_body,
        grid=(E // R,),
        in_specs=[
            pl.BlockSpec((R, ED), lambda i: (i, 0)),
            pl.BlockSpec((ED, H), lambda i: (0, 0)),
            pl.BlockSpec((1, H), lambda i: (0, 0)),
        ],
        out_specs=[
            pl.BlockSpec((R, HH), lambda i: (i, 0)),
            pl.BlockSpec((R, HH), lambda i: (i, 0)),
        ],
        out_shape=[jax.ShapeDtypeStruct((E, HH), jnp.float32)] * 2,
    )(edge_attr, we, be.reshape(1, H))


def _mlp_body(hlo_ref, hhi_ref, alo_ref, ahi_ref, w1_ref, b1_ref, w2_ref,
              b2_ref, olo_ref, ohi_ref):
    z = jnp.concatenate(
        [hlo_ref[...] + alo_ref[...], hhi_ref[...] + ahi_ref[...]], axis=1)
    y = jnp.dot(z, w1_ref[...], preferred_element_type=jnp.float32)
    y = jnp.maximum(y + b1_ref[...], 0.0)
    o = jnp.dot(y, w2_ref[...], preferred_element_type=jnp.float32)
    o = jnp.maximum(o + b2_ref[...], 0.0)
    olo_ref[...] = o[:, :HH]
    ohi_ref[...] = o[:, HH:]


def _mlp(h_lo, h_hi, a_lo, a_hi, w1, b1, w2, b2):
    R = 400
    blk = pl.BlockSpec((R, HH), lambda i: (i, 0))
    wblk = pl.BlockSpec((H, H), lambda i: (0, 0))
    bblk = pl.BlockSpec((1, H), lambda i: (0, 0))
    return pl.pallas_call(
        _mlp_body,
        grid=(N // R,),
        in_specs=[blk, blk, blk, blk, wblk, bblk, wblk, bblk],
        out_specs=[blk, blk],
        out_shape=[jax.ShapeDtypeStruct((N, HH), jnp.float32)] * 2,
    )(h_lo, h_hi, a_lo, a_hi, w1, b1.reshape(1, H), w2, b2.reshape(1, H))


def _maxpool_body(b_ref, h1l, h1h, h2l, h2h, h3l, h3h, out_ref, cnt_ref):
    i = pl.program_id(0)

    @pl.when(i == 0)
    def _():
        out_ref[...] = jnp.full((G, JK), _NEG_INF, jnp.float32)
        cnt_ref[...] = jnp.zeros((G, 128), jnp.float32)

    bvec = b_ref[...]
    jkb = jnp.concatenate(
        [h1l[...], h1h[...], h2l[...], h2h[...], h3l[...], h3h[...]], axis=1)
    for g in range(G):
        mask = bvec == g

        @pl.when(jnp.any(mask))
        def _(g=g):
            m = jnp.max(jnp.where(mask, jkb, _NEG_INF), axis=0)
            out_ref[g, :] = jnp.maximum(out_ref[g, :], m)
            cnt_ref[g, :] = cnt_ref[g, :] + jnp.sum(mask.astype(jnp.float32))


def _maxpool(batch2d, hs):
    R = 400
    blk = pl.BlockSpec((R, HH), lambda i: (i, 0))
    return pl.pallas_call(
        _maxpool_body,
        grid=(N // R,),
        in_specs=[pl.BlockSpec((R, 1), lambda i: (i, 0))] + [blk] * 6,
        out_specs=[pl.BlockSpec((G, JK), lambda i: (0, 0)),
                   pl.BlockSpec((G, 128), lambda i: (0, 0))],
        out_shape=[jax.ShapeDtypeStruct((G, JK), jnp.float32),
                   jax.ShapeDtypeStruct((G, 128), jnp.float32)],
    )(batch2d, *hs)


def _head_body(sum_ref, mx_ref, cnt_ref, w1_ref, b1_ref, w2_ref, b2_ref,
               out_ref):
    s = sum_ref[0] + sum_ref[1]
    mx = mx_ref[...]
    mx = jnp.where(mx == _NEG_INF, 0.0, mx)
    cnt = cnt_ref[:, :1]
    mean = s / jnp.maximum(cnt, 1.0)
    z = jnp.concatenate([mean, mx], axis=1)
    y = jnp.dot(z, w1_ref[...], preferred_element_type=jnp.float32)
    y = jnp.maximum(y + b1_ref[...], 0.0)
    o = jnp.dot(y, w2_ref[...], preferred_element_type=jnp.float32)
    out_ref[...] = o + b2_ref[...]


def _head(sum2, mx, cnt2, wc1, bc1, wc2, bc2):
    return pl.pallas_call(
        _head_body,
        out_shape=jax.ShapeDtypeStruct((G, C), jnp.float32),
    )(sum2, mx, cnt2, wc1, bc1.reshape(1, H), wc2, bc2.reshape(1, C))


@functools.partial(
    pl.kernel,
    out_type=[jax.ShapeDtypeStruct((N, HH), jnp.float32),
              jax.ShapeDtypeStruct((N, HH), jnp.float32)],
    mesh=_SC_MESH,
    scratch_types=[
        [pltpu.VMEM((2, K), jnp.int32) for _ in range(2)],
        [pltpu.VMEM((K,), jnp.int32) for _ in range(2)],
        [pltpu.VMEM((K, HH), jnp.float32) for _ in range(2)],
        [pltpu.VMEM((K, HH), jnp.float32) for _ in range(2)],
        pltpu.VMEM_SHARED((N, HH), jnp.float32),
        [pltpu.SemaphoreType.DMA for _ in range(2)],
        [pltpu.SemaphoreType.DMA for _ in range(2)],
        [pltpu.SemaphoreType.DMA for _ in range(2)],
        [pltpu.SemaphoreType.DMA for _ in range(2)],
    ],
)
def _edge_aggr(ei_hbm, h_lo, h_hi, ea_lo, ea_hi, zeros_hbm,
               out_lo, out_hi, idx_bufs, dst_bufs, h_bufs, ea_bufs, aggr_sh,
               gsems, esems, ssems, isems):
    c = lax.axis_index("c")
    s = lax.axis_index("s")
    row0 = s * ZROWS

    pltpu.sync_copy(zeros_hbm, aggr_sh.at[pl.ds(row0, ZROWS)])

    @pl.when(s == NS - 1)
    def _():
        pltpu.sync_copy(zeros_hbm.at[pl.ds(0, ZTAIL)],
                        aggr_sh.at[pl.ds(NS * ZROWS, ZTAIL)])

    plsc.subcore_barrier()

    def run(h_ref, ea_ref):

        def scatter_wait(b):
            pltpu.make_async_copy(h_bufs[b], aggr_sh.at[dst_bufs[b]],
                                  ssems[b]).wait()

        def idx_and_fire(j, b):
            base = (s + j * NS) * K
            idx_cp = pltpu.async_copy(ei_hbm.at[s + j * NS], idx_bufs[b],
                                      isems[b])

            @pl.when(j >= 2)
            def _():
                scatter_wait(b)

            idx_cp.wait()
            pltpu.async_copy(h_ref.at[idx_bufs[b].at[0]], h_bufs[b],
                             gsems[b])
            pltpu.async_copy(ea_ref.at[pl.ds(base, K)], ea_bufs[b],
                             esems[b])

        idx_and_fire(0, 0)

        def pair_body(p, carry):
            for b in range(2):
                j = 2 * p + b

                @pl.when(j + 1 < CPT)
                def _(b=b, j=j):
                    idx_and_fire(j + 1, 1 - b)

                base = (s + j * NS) * K
                pltpu.make_async_copy(h_ref.at[idx_bufs[b].at[0]],
                                      h_bufs[b], gsems[b]).wait()
                pltpu.make_async_copy(ea_ref.at[pl.ds(base, K)],
                                      ea_bufs[b], esems[b]).wait()

                def row_body(r2, rc):
                    for u in range(2):
                        r = r2 * 2 + u
                        for jj in range(HH // 16):
                            sl = pl.ds(jj * 16, 16)
                            h_bufs[b][r, sl] = jnp.maximum(
                                h_bufs[b][r, sl] + ea_bufs[b][r, sl], 0.0)
                    return rc

                lax.fori_loop(0, K // 2, row_body, 0)
                for q in range(K // 16):
                    qsl = pl.ds(q * 16, 16)
                    dst_bufs[b][qsl] = idx_bufs[b][1, qsl]
                pltpu.async_copy(h_bufs[b], aggr_sh.at[dst_bufs[b]],
                                 ssems[b], add=True)
            return carry

        lax.fori_loop(0, CPT // 2, pair_body, 0)
        scatter_wait(0)
        scatter_wait(1)

    @pl.when(c == 0)
    def _():
        run(h_lo, ea_lo)

    @pl.when(c == 1)
    def _():
        run(h_hi, ea_hi)

    plsc.subcore_barrier()

    def copy_out(out_ref):
        pltpu.sync_copy(aggr_sh.at[pl.ds(row0, ZROWS)],
                        out_ref.at[pl.ds(row0, ZROWS)])

        @pl.when(s == NS - 1)
        def _():
            pltpu.sync_copy(aggr_sh.at[pl.ds(NS * ZROWS, ZTAIL)],
                            out_ref.at[pl.ds(NS * ZROWS, ZTAIL)])

    @pl.when(c == 0)
    def _():
        copy_out(out_lo)

    @pl.when(c == 1)
    def _():
        copy_out(out_hi)


_POOL_CH = 16
_POOL_NCHUNK = N // _POOL_CH


@functools.partial(
    pl.kernel,
    out_type=jax.ShapeDtypeStruct((2, G, JK), jnp.float32),
    mesh=_SC_MESH,
    scratch_types=[
        [pltpu.VMEM((_POOL_CH, HH), jnp.float32) for _ in range(6)],
        pltpu.VMEM((_POOL_CH,), jnp.int32),
        [pltpu.VMEM_SHARED((G, HH), jnp.float32) for _ in range(6)],
    ],
)
def _pool(batch_hbm, h1_lo, h1_hi, h2_lo, h2_hi, h3_lo, h3_hi, z128_hbm,
          out_sum, rbufs, bbuf, ssum):
    c = lax.axis_index("c")
    s = lax.axis_index("s")
    w = c * NS + s
    h_refs = (h1_lo, h1_hi, h2_lo, h2_hi, h3_lo, h3_hi)

    @pl.when(s == 0)
    def _():
        for a in range(6):
            pltpu.sync_copy(z128_hbm, ssum[a])

    plsc.subcore_barrier()

    nchunks = (_POOL_NCHUNK + 2 * NS - 1 - w) // (2 * NS)

    def chunk_body(i, carry):
        base = (w + i * 2 * NS) * _POOL_CH
        pltpu.sync_copy(batch_hbm.at[pl.ds(base, _POOL_CH)], bbuf)
        for a in range(6):
            pltpu.sync_copy(h_refs[a].at[pl.ds(base, _POOL_CH)], rbufs[a])
            pltpu.sync_copy(rbufs[a], ssum[a].at[bbuf], add=True)
        return carry

    lax.fori_loop(0, nchunks, chunk_body, 0)
    plsc.subcore_barrier()

    for a in range(6):

        @pl.when(s == a)
        def _(a=a):
            pltpu.sync_copy(ssum[a], out_sum.at[c, :, pl.ds(a * HH, HH)])


def kernel(x, edge_index, edge_attr, batch, W_in, b_in, We, be, W1, b1, g1,
           bt1, W2, b2, g2, bt2, Wc1, bc1, gc, btc, Wc2, bc2):
    isr = 1.0 / jnp.sqrt(jnp.float32(1.0 + 1e-5))
    s1 = g1 * isr
    w1f = W1 * s1[:, None, :]
    b1f = b1 * s1 + bt1
    s2 = g2 * isr
    w2f = W2 * s2[:, None, :]
    b2f = b2 * s2 + bt2
    sc = gc * isr
    wc1f = Wc1 * sc[None, :]
    bc1f = bc1 * sc + btc

    ei_chunks = jnp.transpose(edge_index.reshape(2, CHUNKS, K), (1, 0, 2))
    zeros = jnp.zeros((ZROWS, HH), jnp.float32)
    z128 = jnp.zeros((G, HH), jnp.float32)

    h_lo, h_hi = _input_proj(x, W_in, b_in)
    hs = []
    for l in range(NLAYERS):
        ea_lo, ea_hi = _ea_proj(edge_attr, We[l], be[l])
        a_lo, a_hi = _edge_aggr(ei_chunks, h_lo, h_hi, ea_lo, ea_hi, zeros)
        h_lo, h_hi = _mlp(h_lo, h_hi, a_lo, a_hi, w1f[l], b1f[l], w2f[l],
                          b2f[l])
        hs += [h_lo, h_hi]

    sum2 = _pool(batch, *hs, z128)
    mx, cnt = _maxpool(batch.reshape(N, 1), hs)
    return _head(sum2, mx, cnt, wc1f, bc1f, Wc2, bc2)


# speedup vs baseline: 3.7067x; 1.0222x over previous
import functools

import jax
import jax.numpy as jnp
from jax import lax
from jax.experimental import pallas as pl
from jax.experimental.pallas import tpu as pltpu
from jax.experimental.pallas import tpu_sc as plsc

N = 10000
E = 320000
D_IN = 128
H = 256
HH = 128
ED = 16
NLAYERS = 3
G = 64
C = 10
JK = H * NLAYERS

NS = 16
K = 80
CHUNKS = E // K
CPT = CHUNKS // NS
ZROWS = 624
ZTAIL = N - NS * ZROWS

_SC_MESH = plsc.VectorSubcoreMesh(core_axis_name="c", subcore_axis_name="s")
_NEG_INF = float("-inf")


def _proj_body(x_ref, w_ref, b_ref, lo_ref, hi_ref):
    acc = jnp.dot(x_ref[...], w_ref[...], preferred_element_type=jnp.float32)
    acc = acc + b_ref[...]
    lo_ref[...] = acc[:, :HH]
    hi_ref[...] = acc[:, HH:]


def _input_proj(x, w, b):
    R = 400
    return pl.pallas_call(
        _proj_body,
        grid=(N // R,),
        in_specs=[
            pl.BlockSpec((R, D_IN), lambda i: (i, 0)),
            pl.BlockSpec((D_IN, H), lambda i: (0, 0)),
            pl.BlockSpec((1, H), lambda i: (0, 0)),
        ],
        out_specs=[
            pl.BlockSpec((R, HH), lambda i: (i, 0)),
            pl.BlockSpec((R, HH), lambda i: (i, 0)),
        ],
        out_shape=[jax.ShapeDtypeStruct((N, HH), jnp.float32)] * 2,
    )(x, w, b.reshape(1, H))


def _ea_proj(edge_attr, we, be):
    R = 2000
    return pl.pallas_call(
        _proj_body,
        grid=(E // R,),
        in_specs=[
            pl.BlockSpec((R, ED), lambda i: (i, 0)),
            pl.BlockSpec((ED, H), lambda i: (0, 0)),
            pl.BlockSpec((1, H), lambda i: (0, 0)),
        ],
        out_specs=[
            pl.BlockSpec((R, HH), lambda i: (i, 0)),
            pl.BlockSpec((R, HH), lambda i: (i, 0)),
        ],
        out_shape=[jax.ShapeDtypeStruct((E, HH), jnp.float32)] * 2,
    )(edge_attr, we, be.reshape(1, H))


def _mlp_body(hlo_ref, hhi_ref, alo_ref, ahi_ref, w1_ref, b1_ref, w2_ref,
              b2_ref, olo_ref, ohi_ref):
    z = jnp.concatenate(
        [hlo_ref[...] + alo_ref[...], hhi_ref[...] + ahi_ref[...]], axis=1)
    y = jnp.dot(z, w1_ref[...], preferred_element_type=jnp.float32)
    y = jnp.maximum(y + b1_ref[...], 0.0)
    o = jnp.dot(y, w2_ref[...], preferred_element_type=jnp.float32)
    o = jnp.maximum(o + b2_ref[...], 0.0)
    olo_ref[...] = o[:, :HH]
    ohi_ref[...] = o[:, HH:]


def _mlp(h_lo, h_hi, a_lo, a_hi, w1, b1, w2, b2):
    R = 400
    blk = pl.BlockSpec((R, HH), lambda i: (i, 0))
    wblk = pl.BlockSpec((H, H), lambda i: (0, 0))
    bblk = pl.BlockSpec((1, H), lambda i: (0, 0))
    return pl.pallas_call(
        _mlp_body,
        grid=(N // R,),
        in_specs=[blk, blk, blk, blk, wblk, bblk, wblk, bblk],
        out_specs=[blk, blk],
        out_shape=[jax.ShapeDtypeStruct((N, HH), jnp.float32)] * 2,
    )(h_lo, h_hi, a_lo, a_hi, w1, b1.reshape(1, H), w2, b2.reshape(1, H))


def _maxpool_body(b_ref, h1l, h1h, h2l, h2h, h3l, h3h, out_ref, cnt_ref):
    i = pl.program_id(0)

    @pl.when(i == 0)
    def _():
        out_ref[...] = jnp.full((G, JK), _NEG_INF, jnp.float32)
        cnt_ref[...] = jnp.zeros((G, 128), jnp.float32)

    bvec = b_ref[...]
    jkb = jnp.concatenate(
        [h1l[...], h1h[...], h2l[...], h2h[...], h3l[...], h3h[...]], axis=1)
    for g in range(G):
        mask = bvec == g

        @pl.when(jnp.any(mask))
        def _(g=g):
            m = jnp.max(jnp.where(mask, jkb, _NEG_INF), axis=0)
            out_ref[g, :] = jnp.maximum(out_ref[g, :], m)
            cnt_ref[g, :] = cnt_ref[g, :] + jnp.sum(mask.astype(jnp.float32))


def _maxpool(batch2d, hs):
    R = 400
    blk = pl.BlockSpec((R, HH), lambda i: (i, 0))
    return pl.pallas_call(
        _maxpool_body,
        grid=(N // R,),
        in_specs=[pl.BlockSpec((R, 1), lambda i: (i, 0))] + [blk] * 6,
        out_specs=[pl.BlockSpec((G, JK), lambda i: (0, 0)),
                   pl.BlockSpec((G, 128), lambda i: (0, 0))],
        out_shape=[jax.ShapeDtypeStruct((G, JK), jnp.float32),
                   jax.ShapeDtypeStruct((G, 128), jnp.float32)],
    )(batch2d, *hs)


def _head_body(sum_ref, mx_ref, cnt_ref, w1_ref, b1_ref, w2_ref, b2_ref,
               out_ref):
    s = sum_ref[0] + sum_ref[1]
    mx = mx_ref[...]
    mx = jnp.where(mx == _NEG_INF, 0.0, mx)
    cnt = cnt_ref[:, :1]
    mean = s / jnp.maximum(cnt, 1.0)
    z = jnp.concatenate([mean, mx], axis=1)
    y = jnp.dot(z, w1_ref[...], preferred_element_type=jnp.float32)
    y = jnp.maximum(y + b1_ref[...], 0.0)
    o = jnp.dot(y, w2_ref[...], preferred_element_type=jnp.float32)
    out_ref[...] = o + b2_ref[...]


def _head(sum2, mx, cnt2, wc1, bc1, wc2, bc2):
    return pl.pallas_call(
        _head_body,
        out_shape=jax.ShapeDtypeStruct((G, C), jnp.float32),
    )(sum2, mx, cnt2, wc1, bc1.reshape(1, H), wc2, bc2.reshape(1, C))


@functools.partial(
    pl.kernel,
    out_type=[jax.ShapeDtypeStruct((N, HH), jnp.float32),
              jax.ShapeDtypeStruct((N, HH), jnp.float32)],
    mesh=_SC_MESH,
    scratch_types=[
        [pltpu.VMEM((2, K), jnp.int32) for _ in range(2)],
        [pltpu.VMEM((K,), jnp.int32) for _ in range(2)],
        [pltpu.VMEM((K, HH), jnp.float32) for _ in range(2)],
        [pltpu.VMEM((K, HH), jnp.float32) for _ in range(2)],
        pltpu.VMEM_SHARED((N, HH), jnp.float32),
        [pltpu.SemaphoreType.DMA for _ in range(2)],
        [pltpu.SemaphoreType.DMA for _ in range(2)],
        [pltpu.SemaphoreType.DMA for _ in range(2)],
        [pltpu.SemaphoreType.DMA for _ in range(2)],
    ],
)
def _edge_aggr(ei_hbm, h_lo, h_hi, ea_lo, ea_hi, zeros_hbm,
               out_lo, out_hi, idx_bufs, dst_bufs, h_bufs, ea_bufs, aggr_sh,
               gsems, esems, ssems, isems):
    c = lax.axis_index("c")
    s = lax.axis_index("s")
    row0 = s * ZROWS

    pltpu.sync_copy(zeros_hbm, aggr_sh.at[pl.ds(row0, ZROWS)])

    @pl.when(s == NS - 1)
    def _():
        pltpu.sync_copy(zeros_hbm.at[pl.ds(0, ZTAIL)],
                        aggr_sh.at[pl.ds(NS * ZROWS, ZTAIL)])

    plsc.subcore_barrier()

    def run(h_ref, ea_ref):

        def scatter_wait(b):
            pltpu.make_async_copy(h_bufs[b], aggr_sh.at[dst_bufs[b]],
                                  ssems[b]).wait()

        def dst_copy(b):
            for q in range(K // 16):
                qsl = pl.ds(q * 16, 16)
                dst_bufs[b][qsl] = idx_bufs[b][1, qsl]

        def idx_issue(j, b):
            pltpu.async_copy(ei_hbm.at[s + j * NS], idx_bufs[b], isems[b])

        def idx_wait(j, b):
            pltpu.make_async_copy(ei_hbm.at[s + j * NS], idx_bufs[b],
                                  isems[b]).wait()

        def fire(j, b):
            base = (s + j * NS) * K
            pltpu.async_copy(h_ref.at[idx_bufs[b].at[0]], h_bufs[b],
                             gsems[b])
            pltpu.async_copy(ea_ref.at[pl.ds(base, K)], ea_bufs[b],
                             esems[b])

        idx_issue(0, 0)
        idx_wait(0, 0)
        fire(0, 0)
        idx_issue(1, 1)

        def pair_body(p, carry):
            for b in range(2):
                j = 2 * p + b
                dst_copy(b)

                @pl.when(j + 1 < CPT)
                def _(b=b, j=j):
                    idx_wait(j + 1, 1 - b)

                    @pl.when(j >= 1)
                    def _(b=b):
                        scatter_wait(1 - b)

                    fire(j + 1, 1 - b)

                base = (s + j * NS) * K
                pltpu.make_async_copy(h_ref.at[idx_bufs[b].at[0]],
                                      h_bufs[b], gsems[b]).wait()
                pltpu.make_async_copy(ea_ref.at[pl.ds(base, K)],
                                      ea_bufs[b], esems[b]).wait()

                @pl.when(j + 2 < CPT)
                def _(b=b, j=j):
                    idx_issue(j + 2, b)

                def row_body(r2, rc):
                    for u in range(2):
                        r = r2 * 2 + u
                        for jj in range(HH // 16):
                            sl = pl.ds(jj * 16, 16)
                            h_bufs[b][r, sl] = jnp.maximum(
                                h_bufs[b][r, sl] + ea_bufs[b][r, sl], 0.0)
                    return rc

                lax.fori_loop(0, K // 2, row_body, 0)
                pltpu.async_copy(h_bufs[b], aggr_sh.at[dst_bufs[b]],
                                 ssems[b], add=True)
            return carry

        lax.fori_loop(0, CPT // 2, pair_body, 0)
        scatter_wait(0)
        scatter_wait(1)

    @pl.when(c == 0)
    def _():
        run(h_lo, ea_lo)

    @pl.when(c == 1)
    def _():
        run(h_hi, ea_hi)

    plsc.subcore_barrier()

    def copy_out(out_ref):
        pltpu.sync_copy(aggr_sh.at[pl.ds(row0, ZROWS)],
                        out_ref.at[pl.ds(row0, ZROWS)])

        @pl.when(s == NS - 1)
        def _():
            pltpu.sync_copy(aggr_sh.at[pl.ds(NS * ZROWS, ZTAIL)],
                            out_ref.at[pl.ds(NS * ZROWS, ZTAIL)])

    @pl.when(c == 0)
    def _():
        copy_out(out_lo)

    @pl.when(c == 1)
    def _():
        copy_out(out_hi)


_POOL_CH = 16
_POOL_NCHUNK = N // _POOL_CH


@functools.partial(
    pl.kernel,
    out_type=jax.ShapeDtypeStruct((2, G, JK), jnp.float32),
    mesh=_SC_MESH,
    scratch_types=[
        [pltpu.VMEM((_POOL_CH, HH), jnp.float32) for _ in range(6)],
        pltpu.VMEM((_POOL_CH,), jnp.int32),
        [pltpu.VMEM_SHARED((G, HH), jnp.float32) for _ in range(6)],
    ],
)
def _pool(batch_hbm, h1_lo, h1_hi, h2_lo, h2_hi, h3_lo, h3_hi, z128_hbm,
          out_sum, rbufs, bbuf, ssum):
    c = lax.axis_index("c")
    s = lax.axis_index("s")
    w = c * NS + s
    h_refs = (h1_lo, h1_hi, h2_lo, h2_hi, h3_lo, h3_hi)

    @pl.when(s == 0)
    def _():
        for a in range(6):
            pltpu.sync_copy(z128_hbm, ssum[a])

    plsc.subcore_barrier()

    nchunks = (_POOL_NCHUNK + 2 * NS - 1 - w) // (2 * NS)

    def chunk_body(i, carry):
        base = (w + i * 2 * NS) * _POOL_CH
        pltpu.sync_copy(batch_hbm.at[pl.ds(base, _POOL_CH)], bbuf)
        for a in range(6):
            pltpu.sync_copy(h_refs[a].at[pl.ds(base, _POOL_CH)], rbufs[a])
            pltpu.sync_copy(rbufs[a], ssum[a].at[bbuf], add=True)
        return carry

    lax.fori_loop(0, nchunks, chunk_body, 0)
    plsc.subcore_barrier()

    for a in range(6):

        @pl.when(s == a)
        def _(a=a):
            pltpu.sync_copy(ssum[a], out_sum.at[c, :, pl.ds(a * HH, HH)])


def kernel(x, edge_index, edge_attr, batch, W_in, b_in, We, be, W1, b1, g1,
           bt1, W2, b2, g2, bt2, Wc1, bc1, gc, btc, Wc2, bc2):
    isr = 1.0 / jnp.sqrt(jnp.float32(1.0 + 1e-5))
    s1 = g1 * isr
    w1f = W1 * s1[:, None, :]
    b1f = b1 * s1 + bt1
    s2 = g2 * isr
    w2f = W2 * s2[:, None, :]
    b2f = b2 * s2 + bt2
    sc = gc * isr
    wc1f = Wc1 * sc[None, :]
    bc1f = bc1 * sc + btc

    ei_chunks = jnp.transpose(edge_index.reshape(2, CHUNKS, K), (1, 0, 2))
    zeros = jnp.zeros((ZROWS, HH), jnp.float32)
    z128 = jnp.zeros((G, HH), jnp.float32)

    h_lo, h_hi = _input_proj(x, W_in, b_in)
    hs = []
    for l in range(NLAYERS):
        ea_lo, ea_hi = _ea_proj(edge_attr, We[l], be[l])
        a_lo, a_hi = _edge_aggr(ei_chunks, h_lo, h_hi, ea_lo, ea_hi, zeros)
        h_lo, h_hi = _mlp(h_lo, h_hi, a_lo, a_hi, w1f[l], b1f[l], w2f[l],
                          b2f[l])
        hs += [h_lo, h_hi]

    sum2 = _pool(batch, *hs, z128)
    mx, cnt = _maxpool(batch.reshape(N, 1), hs)
    return _head(sum2, mx, cnt, wc1f, bc1f, Wc2, bc2)


# speedup vs baseline: 3.7075x; 1.0002x over previous
import functools

import jax
import jax.numpy as jnp
from jax import lax
from jax.experimental import pallas as pl
from jax.experimental.pallas import tpu as pltpu
from jax.experimental.pallas import tpu_sc as plsc

N = 10000
E = 320000
D_IN = 128
H = 256
HH = 128
ED = 16
NLAYERS = 3
G = 64
C = 10
JK = H * NLAYERS

NS = 16
K = 80
CHUNKS = E // K
CPT = CHUNKS // NS
ZROWS = 624
ZTAIL = N - NS * ZROWS

_SC_MESH = plsc.VectorSubcoreMesh(core_axis_name="c", subcore_axis_name="s")
_NEG_INF = float("-inf")


def _proj_body(x_ref, w_ref, b_ref, lo_ref, hi_ref):
    acc = jnp.dot(x_ref[...], w_ref[...], preferred_element_type=jnp.float32)
    acc = acc + b_ref[...]
    lo_ref[...] = acc[:, :HH]
    hi_ref[...] = acc[:, HH:]


def _input_proj(x, w, b):
    R = 400
    return pl.pallas_call(
        _proj_body,
        grid=(N // R,),
        in_specs=[
            pl.BlockSpec((R, D_IN), lambda i: (i, 0)),
            pl.BlockSpec((D_IN, H), lambda i: (0, 0)),
            pl.BlockSpec((1, H), lambda i: (0, 0)),
        ],
        out_specs=[
            pl.BlockSpec((R, HH), lambda i: (i, 0)),
            pl.BlockSpec((R, HH), lambda i: (i, 0)),
        ],
        out_shape=[jax.ShapeDtypeStruct((N, HH), jnp.float32)] * 2,
    )(x, w, b.reshape(1, H))


def _ea_proj(edge_attr, we, be):
    R = 2000
    return pl.pallas_call(
        _proj_body,
        grid=(E // R,),
        in_specs=[
            pl.BlockSpec((R, ED), lambda i: (i, 0)),
            pl.BlockSpec((ED, H), lambda i: (0, 0)),
            pl.BlockSpec((1, H), lambda i: (0, 0)),
        ],
        out_specs=[
            pl.BlockSpec((R, HH), lambda i: (i, 0)),
            pl.BlockSpec((R, HH), lambda i: (i, 0)),
        ],
        out_shape=[jax.ShapeDtypeStruct((E, HH), jnp.float32)] * 2,
    )(edge_attr, we, be.reshape(1, H))


def _mlp_body(hlo_ref, hhi_ref, alo_ref, ahi_ref, w1_ref, b1_ref, w2_ref,
              b2_ref, olo_ref, ohi_ref):
    z = jnp.concatenate(
        [hlo_ref[...] + alo_ref[...], hhi_ref[...] + ahi_ref[...]], axis=1)
    y = jnp.dot(z, w1_ref[...], preferred_element_type=jnp.float32)
    y = jnp.maximum(y + b1_ref[...], 0.0)
    o = jnp.dot(y, w2_ref[...], preferred_element_type=jnp.float32)
    o = jnp.maximum(o + b2_ref[...], 0.0)
    olo_ref[...] = o[:, :HH]
    ohi_ref[...] = o[:, HH:]


def _mlp(h_lo, h_hi, a_lo, a_hi, w1, b1, w2, b2):
    R = 400
    blk = pl.BlockSpec((R, HH), lambda i: (i, 0))
    wblk = pl.BlockSpec((H, H), lambda i: (0, 0))
    bblk = pl.BlockSpec((1, H), lambda i: (0, 0))
    return pl.pallas_call(
        _mlp_body,
        grid=(N // R,),
        in_specs=[blk, blk, blk, blk, wblk, bblk, wblk, bblk],
        out_specs=[blk, blk],
        out_shape=[jax.ShapeDtypeStruct((N, HH), jnp.float32)] * 2,
    )(h_lo, h_hi, a_lo, a_hi, w1, b1.reshape(1, H), w2, b2.reshape(1, H))


def _maxpool_body(b_ref, h1l, h1h, h2l, h2h, h3l, h3h, out_ref, cnt_ref):
    i = pl.program_id(0)

    @pl.when(i == 0)
    def _():
        out_ref[...] = jnp.full((G, JK), _NEG_INF, jnp.float32)
        cnt_ref[...] = jnp.zeros((G, 128), jnp.float32)

    bvec = b_ref[...]
    jkb = jnp.concatenate(
        [h1l[...], h1h[...], h2l[...], h2h[...], h3l[...], h3h[...]], axis=1)
    for g in range(G):
        mask = bvec == g

        @pl.when(jnp.any(mask))
        def _(g=g):
            m = jnp.max(jnp.where(mask, jkb, _NEG_INF), axis=0)
            out_ref[g, :] = jnp.maximum(out_ref[g, :], m)
            cnt_ref[g, :] = cnt_ref[g, :] + jnp.sum(mask.astype(jnp.float32))


def _maxpool(batch2d, hs):
    R = 400
    blk = pl.BlockSpec((R, HH), lambda i: (i, 0))
    return pl.pallas_call(
        _maxpool_body,
        grid=(N // R,),
        in_specs=[pl.BlockSpec((R, 1), lambda i: (i, 0))] + [blk] * 6,
        out_specs=[pl.BlockSpec((G, JK), lambda i: (0, 0)),
                   pl.BlockSpec((G, 128), lambda i: (0, 0))],
        out_shape=[jax.ShapeDtypeStruct((G, JK), jnp.float32),
                   jax.ShapeDtypeStruct((G, 128), jnp.float32)],
    )(batch2d, *hs)


def _head_body(sum_ref, mx_ref, cnt_ref, w1_ref, b1_ref, w2_ref, b2_ref,
               out_ref):
    s = sum_ref[0] + sum_ref[1]
    mx = mx_ref[...]
    mx = jnp.where(mx == _NEG_INF, 0.0, mx)
    cnt = cnt_ref[:, :1]
    mean = s / jnp.maximum(cnt, 1.0)
    z = jnp.concatenate([mean, mx], axis=1)
    y = jnp.dot(z, w1_ref[...], preferred_element_type=jnp.float32)
    y = jnp.maximum(y + b1_ref[...], 0.0)
    o = jnp.dot(y, w2_ref[...], preferred_element_type=jnp.float32)
    out_ref[...] = o + b2_ref[...]


def _head(sum2, mx, cnt2, wc1, bc1, wc2, bc2):
    return pl.pallas_call(
        _head_body,
        out_shape=jax.ShapeDtypeStruct((G, C), jnp.float32),
    )(sum2, mx, cnt2, wc1, bc1.reshape(1, H), wc2, bc2.reshape(1, C))


@functools.partial(
    pl.kernel,
    out_type=[jax.ShapeDtypeStruct((N, HH), jnp.float32),
              jax.ShapeDtypeStruct((N, HH), jnp.float32)],
    mesh=_SC_MESH,
    scratch_types=[
        [pltpu.VMEM((2, K), jnp.int32) for _ in range(2)],
        [pltpu.VMEM((K,), jnp.int32) for _ in range(2)],
        [pltpu.VMEM((K, HH), jnp.float32) for _ in range(2)],
        [pltpu.VMEM((K, HH), jnp.float32) for _ in range(2)],
        pltpu.VMEM_SHARED((N, HH), jnp.float32),
        [pltpu.SemaphoreType.DMA for _ in range(2)],
        [pltpu.SemaphoreType.DMA for _ in range(2)],
        [pltpu.SemaphoreType.DMA for _ in range(2)],
        [pltpu.SemaphoreType.DMA for _ in range(2)],
    ],
)
def _edge_aggr(ei_hbm, h_lo, h_hi, ea_lo, ea_hi, zeros_hbm,
               out_lo, out_hi, idx_bufs, dst_bufs, h_bufs, ea_bufs, aggr_sh,
               gsems, esems, ssems, isems):
    c = lax.axis_index("c")
    s = lax.axis_index("s")
    row0 = s * ZROWS

    pltpu.sync_copy(zeros_hbm, aggr_sh.at[pl.ds(row0, ZROWS)])

    @pl.when(s == NS - 1)
    def _():
        pltpu.sync_copy(zeros_hbm.at[pl.ds(0, ZTAIL)],
                        aggr_sh.at[pl.ds(NS * ZROWS, ZTAIL)])

    plsc.subcore_barrier()

    def run(h_ref, ea_ref):

        def scatter_wait(b):
            pltpu.make_async_copy(h_bufs[b], aggr_sh.at[dst_bufs[b]],
                                  ssems[b]).wait()

        def dst_copy(b):
            for q in range(K // 16):
                qsl = pl.ds(q * 16, 16)
                dst_bufs[b][qsl] = idx_bufs[b][1, qsl]

        def idx_issue(j, b):
            pltpu.async_copy(ei_hbm.at[s + j * NS], idx_bufs[b], isems[b])

        def idx_wait(j, b):
            pltpu.make_async_copy(ei_hbm.at[s + j * NS], idx_bufs[b],
                                  isems[b]).wait()

        def fire(j, b):
            base = (s + j * NS) * K
            pltpu.async_copy(h_ref.at[idx_bufs[b].at[0]], h_bufs[b],
                             gsems[b])
            pltpu.async_copy(ea_ref.at[pl.ds(base, K)], ea_bufs[b],
                             esems[b])

        idx_issue(0, 0)
        idx_wait(0, 0)
        fire(0, 0)
        idx_issue(1, 1)

        def pair_body(p, carry):
            for b in range(2):
                j = 2 * p + b
                dst_copy(b)

                @pl.when(j + 1 < CPT)
                def _(b=b, j=j):
                    idx_wait(j + 1, 1 - b)

                    @pl.when(j >= 1)
                    def _(b=b):
                        scatter_wait(1 - b)

                    fire(j + 1, 1 - b)

                base = (s + j * NS) * K
                pltpu.make_async_copy(h_ref.at[idx_bufs[b].at[0]],
                                      h_bufs[b], gsems[b]).wait()
                pltpu.make_async_copy(ea_ref.at[pl.ds(base, K)],
                                      ea_bufs[b], esems[b]).wait()

                @pl.when(j + 2 < CPT)
                def _(b=b, j=j):
                    idx_issue(j + 2, b)

                def row_body(r4, rc):
                    for u in range(4):
                        r = r4 * 4 + u
                        for jj in range(HH // 16):
                            sl = pl.ds(jj * 16, 16)
                            h_bufs[b][r, sl] = jnp.maximum(
                                h_bufs[b][r, sl] + ea_bufs[b][r, sl], 0.0)
                    return rc

                lax.fori_loop(0, K // 4, row_body, 0)
                pltpu.async_copy(h_bufs[b], aggr_sh.at[dst_bufs[b]],
                                 ssems[b], add=True)
            return carry

        lax.fori_loop(0, CPT // 2, pair_body, 0)
        scatter_wait(0)
        scatter_wait(1)

    @pl.when(c == 0)
    def _():
        run(h_lo, ea_lo)

    @pl.when(c == 1)
    def _():
        run(h_hi, ea_hi)

    plsc.subcore_barrier()

    def copy_out(out_ref):
        pltpu.sync_copy(aggr_sh.at[pl.ds(row0, ZROWS)],
                        out_ref.at[pl.ds(row0, ZROWS)])

        @pl.when(s == NS - 1)
        def _():
            pltpu.sync_copy(aggr_sh.at[pl.ds(NS * ZROWS, ZTAIL)],
                            out_ref.at[pl.ds(NS * ZROWS, ZTAIL)])

    @pl.when(c == 0)
    def _():
        copy_out(out_lo)

    @pl.when(c == 1)
    def _():
        copy_out(out_hi)


_POOL_CH = 16
_POOL_NCHUNK = N // _POOL_CH


@functools.partial(
    pl.kernel,
    out_type=jax.ShapeDtypeStruct((2, G, JK), jnp.float32),
    mesh=_SC_MESH,
    scratch_types=[
        [pltpu.VMEM((_POOL_CH, HH), jnp.float32) for _ in range(6)],
        pltpu.VMEM((_POOL_CH,), jnp.int32),
        [pltpu.VMEM_SHARED((G, HH), jnp.float32) for _ in range(6)],
    ],
)
def _pool(batch_hbm, h1_lo, h1_hi, h2_lo, h2_hi, h3_lo, h3_hi, z128_hbm,
          out_sum, rbufs, bbuf, ssum):
    c = lax.axis_index("c")
    s = lax.axis_index("s")
    w = c * NS + s
    h_refs = (h1_lo, h1_hi, h2_lo, h2_hi, h3_lo, h3_hi)

    @pl.when(s == 0)
    def _():
        for a in range(6):
            pltpu.sync_copy(z128_hbm, ssum[a])

    plsc.subcore_barrier()

    nchunks = (_POOL_NCHUNK + 2 * NS - 1 - w) // (2 * NS)

    def chunk_body(i, carry):
        base = (w + i * 2 * NS) * _POOL_CH
        pltpu.sync_copy(batch_hbm.at[pl.ds(base, _POOL_CH)], bbuf)
        for a in range(6):
            pltpu.sync_copy(h_refs[a].at[pl.ds(base, _POOL_CH)], rbufs[a])
            pltpu.sync_copy(rbufs[a], ssum[a].at[bbuf], add=True)
        return carry

    lax.fori_loop(0, nchunks, chunk_body, 0)
    plsc.subcore_barrier()

    for a in range(6):

        @pl.when(s == a)
        def _(a=a):
            pltpu.sync_copy(ssum[a], out_sum.at[c, :, pl.ds(a * HH, HH)])


def kernel(x, edge_index, edge_attr, batch, W_in, b_in, We, be, W1, b1, g1,
           bt1, W2, b2, g2, bt2, Wc1, bc1, gc, btc, Wc2, bc2):
    isr = 1.0 / jnp.sqrt(jnp.float32(1.0 + 1e-5))
    s1 = g1 * isr
    w1f = W1 * s1[:, None, :]
    b1f = b1 * s1 + bt1
    s2 = g2 * isr
    w2f = W2 * s2[:, None, :]
    b2f = b2 * s2 + bt2
    sc = gc * isr
    wc1f = Wc1 * sc[None, :]
    bc1f = bc1 * sc + btc

    ei_chunks = jnp.transpose(edge_index.reshape(2, CHUNKS, K), (1, 0, 2))
    zeros = jnp.zeros((ZROWS, HH), jnp.float32)
    z128 = jnp.zeros((G, HH), jnp.float32)

    h_lo, h_hi = _input_proj(x, W_in, b_in)
    hs = []
    for l in range(NLAYERS):
        ea_lo, ea_hi = _ea_proj(edge_attr, We[l], be[l])
        a_lo, a_hi = _edge_aggr(ei_chunks, h_lo, h_hi, ea_lo, ea_hi, zeros)
        h_lo, h_hi = _mlp(h_lo, h_hi, a_lo, a_hi, w1f[l], b1f[l], w2f[l],
                          b2f[l])
        hs += [h_lo, h_hi]

    sum2 = _pool(batch, *hs, z128)
    mx, cnt = _maxpool(batch.reshape(N, 1), hs)
    return _head(sum2, mx, cnt, wc1f, bc1f, Wc2, bc2)
